# baseline (speedup 1.0000x reference)
import functools

import jax
import jax.numpy as jnp
from jax.experimental import pallas as pl
from jax.experimental.pallas import tpu as pltpu

_BQ = 256
_BK = 1024
_PAD_VAL = 3e4


def _rank_kernel(sample_ref, positive_ref, gal_ref, out_ref, thresh_ref,
                 acc_ref, *, nk, bk, precision):
    ik = pl.program_id(1)

    @pl.when(ik == 0)
    def _init():
        a = sample_ref[...]
        t = a - positive_ref[...] + 1e-6
        t2 = jnp.sum(t * t, axis=1, keepdims=True)
        a2 = jnp.sum(a * a, axis=1, keepdims=True)
        thresh_ref[...] = jnp.where(t2 >= 1e-12, 0.5 * (t2 - a2), -jnp.inf)
        acc_ref[...] = jnp.zeros_like(acc_ref)

    g = gal_ref[...]
    ones8 = jnp.full((8, g.shape[1]), 0.5, jnp.float32)
    b2h = jax.lax.dot_general(
        ones8, g * g,
        dimension_numbers=(((1,), (1,)), ((), ())),
        preferred_element_type=jnp.float32,
        precision=jax.lax.Precision.HIGHEST)
    ab = jax.lax.dot_general(
        sample_ref[...], g,
        dimension_numbers=(((1,), (1,)), ((), ())),
        preferred_element_type=jnp.float32,
        precision=precision)
    mask = ((b2h[0:1, :] - ab) <= thresh_ref[...]).astype(jnp.int32)
    partial = mask[:, 0:128]
    for c in range(1, bk // 128):
        partial = partial + mask[:, c * 128:(c + 1) * 128]
    acc_ref[...] = acc_ref[...] + partial

    @pl.when(ik == nk - 1)
    def _fin():
        rank = jnp.sum(acc_ref[...], axis=1)
        out_ref[...] = jnp.maximum(rank, 1)


def _ranks(sample_feature, positive_feature, gallery, precision):
    q, d = sample_feature.shape
    k = gallery.shape[0]
    bq = min(_BQ, q)
    bk = min(_BK, k)
    nq = pl.cdiv(q, bq)
    nk = pl.cdiv(k, bk)
    k_pad = nk * bk
    if k_pad != k:
        pad = jnp.full((k_pad - k, d), _PAD_VAL, gallery.dtype)
        gallery = jnp.concatenate([gallery, pad], axis=0)

    kern = functools.partial(_rank_kernel, nk=nk, bk=bk, precision=precision)
    return pl.pallas_call(
        kern,
        grid=(nq, nk),
        in_specs=[
            pl.BlockSpec((bq, d), lambda iq, ik: (iq, 0)),
            pl.BlockSpec((bq, d), lambda iq, ik: (iq, 0)),
            pl.BlockSpec((bk, d), lambda iq, ik: (ik, 0)),
        ],
        out_specs=pl.BlockSpec((bq,), lambda iq, ik: (iq,)),
        out_shape=jax.ShapeDtypeStruct((q,), jnp.int32),
        scratch_shapes=[
            pltpu.VMEM((bq, 1), jnp.float32),
            pltpu.VMEM((bq, 128), jnp.int32),
        ],
        compiler_params=pltpu.CompilerParams(
            dimension_semantics=("arbitrary", "arbitrary")),
    )(sample_feature, positive_feature, gallery)


def kernel(sample_feature, positive_feature, gallery):
    rank = _ranks(sample_feature, positive_feature, gallery,
                  precision=jax.lax.Precision.HIGHEST)
    rank_f = rank.astype(jnp.float32)
    top1 = jnp.mean((rank <= 1).astype(jnp.float32))
    top10 = jnp.mean((rank <= 10).astype(jnp.float32))
    avg = jnp.mean(rank_f)
    return (rank, top1, top10, avg)


# speedup vs baseline: 1.4860x; 1.4860x over previous
import functools

import jax
import jax.numpy as jnp
from jax.experimental import pallas as pl
from jax.experimental.pallas import tpu as pltpu

_BQ = 256
_BK = 1024
_PAD_VAL = 3e4


def _rank_kernel(sample_ref, positive_ref, gal_ref, out_ref, thresh_ref,
                 acc_ref, *, nk, bk, precision):
    ik = pl.program_id(1)

    @pl.when(ik == 0)
    def _init():
        a = sample_ref[...]
        t = a - positive_ref[...] + 1e-6
        t2 = jnp.sum(t * t, axis=1, keepdims=True)
        a2 = jnp.sum(a * a, axis=1, keepdims=True)
        thresh_ref[...] = jnp.where(t2 >= 1e-12, 0.5 * (t2 - a2), -jnp.inf)
        acc_ref[...] = jnp.zeros_like(acc_ref)

    g = gal_ref[...]
    ones8 = jnp.full((8, g.shape[1]), 0.5, jnp.float32)
    b2h = jax.lax.dot_general(
        ones8, g * g,
        dimension_numbers=(((1,), (1,)), ((), ())),
        preferred_element_type=jnp.float32,
        precision=jax.lax.Precision.HIGHEST)
    ab = jax.lax.dot_general(
        sample_ref[...], g,
        dimension_numbers=(((1,), (1,)), ((), ())),
        preferred_element_type=jnp.float32,
        precision=precision)
    mask = ((b2h[0:1, :] - ab) <= thresh_ref[...]).astype(jnp.int32)
    partial = mask[:, 0:128]
    for c in range(1, bk // 128):
        partial = partial + mask[:, c * 128:(c + 1) * 128]
    acc_ref[...] = acc_ref[...] + partial

    @pl.when(ik == nk - 1)
    def _fin():
        rank = jnp.sum(acc_ref[...], axis=1)
        out_ref[...] = jnp.maximum(rank, 1)


def _ranks(sample_feature, positive_feature, gallery, precision):
    q, d = sample_feature.shape
    k = gallery.shape[0]
    bq = min(_BQ, q)
    bk = min(_BK, k)
    nq = pl.cdiv(q, bq)
    nk = pl.cdiv(k, bk)
    k_pad = nk * bk
    if k_pad != k:
        pad = jnp.full((k_pad - k, d), _PAD_VAL, gallery.dtype)
        gallery = jnp.concatenate([gallery, pad], axis=0)

    kern = functools.partial(_rank_kernel, nk=nk, bk=bk, precision=precision)
    return pl.pallas_call(
        kern,
        grid=(nq, nk),
        in_specs=[
            pl.BlockSpec((bq, d), lambda iq, ik: (iq, 0)),
            pl.BlockSpec((bq, d), lambda iq, ik: (iq, 0)),
            pl.BlockSpec((bk, d), lambda iq, ik: (ik, 0)),
        ],
        out_specs=pl.BlockSpec((bq,), lambda iq, ik: (iq,)),
        out_shape=jax.ShapeDtypeStruct((q,), jnp.int32),
        scratch_shapes=[
            pltpu.VMEM((bq, 1), jnp.float32),
            pltpu.VMEM((bq, 128), jnp.int32),
        ],
        compiler_params=pltpu.CompilerParams(
            dimension_semantics=("arbitrary", "arbitrary")),
    )(sample_feature, positive_feature, gallery)


def kernel(sample_feature, positive_feature, gallery):
    rank = _ranks(sample_feature, positive_feature, gallery,
                  precision=jax.lax.Precision.DEFAULT)
    rank_f = rank.astype(jnp.float32)
    top1 = jnp.mean((rank <= 1).astype(jnp.float32))
    top10 = jnp.mean((rank <= 10).astype(jnp.float32))
    avg = jnp.mean(rank_f)
    return (rank, top1, top10, avg)


# speedup vs baseline: 5.6312x; 3.7894x over previous
import functools

import jax
import jax.numpy as jnp
from jax.experimental import pallas as pl
from jax.experimental.pallas import tpu as pltpu

_BK = 4096
_BQ = 1024


def _thr_kernel(sample_ref, positive_ref, thr_ref):
    s = sample_ref[...]
    t = s - positive_ref[...] + 1e-6
    t2 = jnp.sum(t * t, axis=1, keepdims=True)
    a2 = jnp.sum(s * s, axis=1, keepdims=True)
    thr_ref[...] = jnp.where(t2 >= 1e-12, 0.5 * (t2 - a2), -1e30)


def _count_kernel(sample_ref, thr_ref, gal_ref, acc_ref, *, bk, nk, rem):
    j = pl.program_id(1)

    @pl.when(j == 0)
    def _init():
        acc_ref[...] = jnp.zeros_like(acc_ref)

    g = gal_ref[...]
    negh = jnp.full((8, g.shape[1]), -0.5, jnp.float32)
    nb = jax.lax.dot_general(
        negh, g * g,
        dimension_numbers=(((1,), (1,)), ((), ())),
        preferred_element_type=jnp.float32)
    ab = jax.lax.dot_general(
        sample_ref[...], g,
        dimension_numbers=(((1,), (1,)), ((), ())),
        preferred_element_type=jnp.float32)
    z = (ab + nb[0:1, :]) + thr_ref[...]

    def _accum(zv):
        neg = jnp.right_shift(jax.lax.bitcast_convert_type(zv, jnp.int32), 31)
        partial = neg[:, 0:128]
        for c in range(1, bk // 128):
            partial = partial + neg[:, c * 128:(c + 1) * 128]
        acc_ref[...] = acc_ref[...] + partial

    @pl.when(j < nk - 1)
    def _full():
        _accum(z)

    @pl.when(j == nk - 1)
    def _tail():
        col = jax.lax.broadcasted_iota(jnp.int32, z.shape, 1)
        _accum(jnp.where(col < rem, z, -1.0))


def _fin_kernel(acc_ref, out_ref, *, k_pad):
    rank = k_pad + jnp.sum(acc_ref[...], axis=1)
    out_ref[...] = jnp.maximum(rank, 1)


def _ranks(sample_feature, positive_feature, gallery):
    q, d = sample_feature.shape
    k = gallery.shape[0]
    bk = min(_BK, k)
    nk = pl.cdiv(k, bk)
    k_pad = nk * bk
    rem = k - (nk - 1) * bk

    thr = pl.pallas_call(
        _thr_kernel,
        out_shape=jax.ShapeDtypeStruct((q, 1), jnp.float32),
    )(sample_feature, positive_feature)

    bq = _BQ if q % _BQ == 0 else q
    nq = q // bq
    acc = pl.pallas_call(
        functools.partial(_count_kernel, bk=bk, nk=nk, rem=rem),
        grid=(nq, nk),
        in_specs=[
            pl.BlockSpec((bq, d), lambda i, j: (i, 0)),
            pl.BlockSpec((bq, 1), lambda i, j: (i, 0)),
            pl.BlockSpec((bk, d), lambda i, j: (j, 0)),
        ],
        out_specs=pl.BlockSpec((bq, 128), lambda i, j: (i, 0)),
        out_shape=jax.ShapeDtypeStruct((q, 128), jnp.int32),
        compiler_params=pltpu.CompilerParams(
            dimension_semantics=("parallel", "arbitrary")),
    )(sample_feature, thr, gallery)

    return pl.pallas_call(
        functools.partial(_fin_kernel, k_pad=k_pad),
        out_shape=jax.ShapeDtypeStruct((q,), jnp.int32),
    )(acc)


def kernel(sample_feature, positive_feature, gallery):
    rank = _ranks(sample_feature, positive_feature, gallery)
    rank_f = rank.astype(jnp.float32)
    top1 = jnp.mean((rank <= 1).astype(jnp.float32))
    top10 = jnp.mean((rank <= 10).astype(jnp.float32))
    avg = jnp.mean(rank_f)
    return (rank, top1, top10, avg)
